# baseline (speedup 1.0000x reference)
import jax
import jax.numpy as jnp
from jax import lax
from jax.experimental import pallas as pl
from jax.experimental.pallas import tpu as pltpu
from jax.experimental.pallas import tpu_sc as plsc

VOCAB = 1000
BATCH = 16384

_info = plsc.get_sparse_core_info()
_NC, _NS, _L = _info.num_cores, _info.num_subcores, _info.num_lanes
_NW = _NC * _NS
_COLS_PER_W = BATCH // _NW
CHUNK = 128
_NCHUNK = _COLS_PER_W // CHUNK
_VPC = CHUNK // _L
_ZROWS = 192
_SPLIT = 504
_HALVES = ((0, _SPLIT), (_SPLIT, VOCAB - _SPLIT))


def _sc_onehot_t(inp_hbm, lmap_hbm, zeros_hbm, out_hbm, inp_v, lmap_v,
                 zshared, buf_a, buf_b, sem_a, sem_b):
    bufs = (buf_a, buf_b)
    sems = (sem_a, sem_b)
    sid = lax.axis_index("s")
    wid = sid * _NC + lax.axis_index("c")
    base_col = wid * _COLS_PER_W

    @pl.when(sid == 0)
    def _():
        pltpu.sync_copy(zeros_hbm, zshared)

    pltpu.sync_copy(inp_hbm.at[pl.ds(base_col, _COLS_PER_W)], inp_v)
    pltpu.sync_copy(lmap_hbm, lmap_v)
    plsc.subcore_barrier()
    for b, (_, nrows) in zip(bufs, _HALVES):
        for r in range(0, nrows, _ZROWS):
            n = min(_ZROWS, nrows - r)
            pltpu.sync_copy(zshared.at[pl.ds(0, n)], b.at[pl.ds(r, n)])

    lane = lax.iota(jnp.int32, _L)
    ones = jnp.full((_L,), 1, jnp.int32)
    zero = jnp.full((_L,), 0, jnp.int32)
    handles = [None, None]

    def chunk_ids(c, j):
        raw = inp_v[pl.ds(c * CHUNK + j * _L, _L)]
        return plsc.load_gather(lmap_v, [raw])

    def scatter_half(h, c, val):
        lo, nrows = _HALVES[h]
        for j in range(_VPC):
            ids = chunk_ids(c, j)
            if h == 0:
                mask = ids < _SPLIT
                rows = jnp.minimum(ids, _SPLIT - 1)
            else:
                mask = ids >= _SPLIT
                rows = jnp.maximum(ids - _SPLIT, 0)
            plsc.store_scatter(bufs[h], [rows, lane + j * _L], val, mask=mask)

    for c in range(_NCHUNK):
        col = base_col + c * CHUNK
        for h in (0, 1):
            if handles[h] is not None:
                handles[h].wait()
                scatter_half(h, c - 1, zero)
            scatter_half(h, c, ones)
            lo, nrows = _HALVES[h]
            handles[h] = pltpu.async_copy(
                bufs[h], out_hbm.at[pl.ds(lo, nrows), pl.ds(col, CHUNK)],
                sems[h])

    for h in handles:
        h.wait()


def kernel(input, label_map):
    zeros = jnp.zeros((_ZROWS, CHUNK), jnp.int32)
    run = pl.kernel(
        _sc_onehot_t,
        out_type=jax.ShapeDtypeStruct((VOCAB, BATCH), jnp.int32),
        mesh=plsc.VectorSubcoreMesh(core_axis_name="c", subcore_axis_name="s"),
        compiler_params=pltpu.CompilerParams(
            needs_layout_passes=False, use_tc_tiling_on_sc=True),
        scratch_types=[
            pltpu.VMEM((_COLS_PER_W,), jnp.int32),
            pltpu.VMEM((VOCAB,), jnp.int32),
            pltpu.VMEM_SHARED((_ZROWS, CHUNK), jnp.int32),
            pltpu.VMEM((_SPLIT, CHUNK), jnp.int32),
            pltpu.VMEM((VOCAB - _SPLIT, CHUNK), jnp.int32),
            pltpu.SemaphoreType.DMA,
            pltpu.SemaphoreType.DMA,
        ],
    )
    return run(input, label_map, zeros).T


# speedup vs baseline: 1.0372x; 1.0372x over previous
import jax
import jax.numpy as jnp
from jax import lax
from jax.experimental import pallas as pl
from jax.experimental.pallas import tpu as pltpu
from jax.experimental.pallas import tpu_sc as plsc

VOCAB = 1000
BATCH = 16384

_info = plsc.get_sparse_core_info()
_NC, _NS, _L = _info.num_cores, _info.num_subcores, _info.num_lanes
_NW = _NC * _NS
_COLS_PER_W = BATCH // _NW
CHUNK = 128
_NCHUNK = _COLS_PER_W // CHUNK
_VPC = CHUNK // _L
_ZROWS = 192


def _sc_onehot_t(inp_hbm, lmap_hbm, zeros_hbm, out_hbm, inp_v, lmap_v, buf,
                 zshared):
    sid = lax.axis_index("s")
    wid = sid * _NC + lax.axis_index("c")
    base_col = wid * _COLS_PER_W

    @pl.when(sid == 0)
    def _():
        pltpu.sync_copy(zeros_hbm, zshared)

    pltpu.sync_copy(inp_hbm.at[pl.ds(base_col, _COLS_PER_W)], inp_v)
    pltpu.sync_copy(lmap_hbm, lmap_v)
    plsc.subcore_barrier()
    for r in range(0, VOCAB, _ZROWS):
        n = min(_ZROWS, VOCAB - r)
        pltpu.sync_copy(zshared.at[pl.ds(0, n)], buf.at[pl.ds(r, n)])

    lane = lax.iota(jnp.int32, _L)
    ones = jnp.full((_L,), 1, jnp.int32)
    zero = jnp.full((_L,), 0, jnp.int32)

    def chunk_body(c, carry):
        ids = []
        for j in range(_VPC):
            raw = inp_v[pl.ds(c * CHUNK + j * _L, _L)]
            ids.append(plsc.load_gather(lmap_v, [raw]))
        for j in range(_VPC):
            plsc.store_scatter(buf, [ids[j], lane + j * _L], ones)
        col = pl.multiple_of(base_col + c * CHUNK, CHUNK)
        pltpu.sync_copy(buf, out_hbm.at[:, pl.ds(col, CHUNK)])
        for j in range(_VPC):
            plsc.store_scatter(buf, [ids[j], lane + j * _L], zero)
        return carry

    lax.fori_loop(0, _NCHUNK, chunk_body, 0)


def kernel(input, label_map):
    zeros = jnp.zeros((_ZROWS, CHUNK), jnp.int32)
    run = pl.kernel(
        _sc_onehot_t,
        out_type=jax.ShapeDtypeStruct((VOCAB, BATCH), jnp.int32),
        mesh=plsc.VectorSubcoreMesh(core_axis_name="c", subcore_axis_name="s"),
        compiler_params=pltpu.CompilerParams(
            needs_layout_passes=False, use_tc_tiling_on_sc=True),
        scratch_types=[
            pltpu.VMEM((_COLS_PER_W,), jnp.int32),
            pltpu.VMEM((VOCAB,), jnp.int32),
            pltpu.VMEM((VOCAB, CHUNK), jnp.int32),
            pltpu.VMEM_SHARED((_ZROWS, CHUNK), jnp.int32),
        ],
    )
    return run(input, label_map, zeros).T


# speedup vs baseline: 1.0377x; 1.0005x over previous
import jax
import jax.numpy as jnp
from jax import lax
from jax.experimental import pallas as pl
from jax.experimental.pallas import tpu as pltpu
from jax.experimental.pallas import tpu_sc as plsc

VOCAB = 1000
BATCH = 16384

_info = plsc.get_sparse_core_info()
_NC, _NS, _L = _info.num_cores, _info.num_subcores, _info.num_lanes
_NW = _NC * _NS
_COLS_PER_W = BATCH // _NW
CHUNK = 128
_NCHUNK = _COLS_PER_W // CHUNK
_VPC = CHUNK // _L
_ZROWS = 192


def _sc_onehot_t(inp_hbm, lmap_hbm, zeros_hbm, out_hbm, inp_v, lmap_v, buf,
                 zshared):
    sid = lax.axis_index("s")
    wid = sid * _NC + lax.axis_index("c")
    base_col = wid * _COLS_PER_W

    @pl.when(sid == 0)
    def _():
        pltpu.sync_copy(zeros_hbm, zshared)

    pltpu.sync_copy(inp_hbm.at[pl.ds(base_col, _COLS_PER_W)], inp_v)
    pltpu.sync_copy(lmap_hbm, lmap_v)
    plsc.subcore_barrier()
    for r in range(0, VOCAB, _ZROWS):
        n = min(_ZROWS, VOCAB - r)
        pltpu.sync_copy(zshared.at[pl.ds(0, n)], buf.at[pl.ds(r, n)])

    lane = lax.iota(jnp.int32, _L)
    ones = jnp.full((_L,), 1, jnp.int32)
    zero = jnp.full((_L,), 0, jnp.int32)

    def scatter_pass(c, val):
        def body(j, carry):
            raw = inp_v[pl.ds(c * CHUNK + j * _L, _L)]
            ids = plsc.load_gather(lmap_v, [raw])
            plsc.store_scatter(buf, [ids, lane + j * _L], val)
            return carry
        lax.fori_loop(0, _VPC, body, 0)

    def chunk_body(c, carry):
        scatter_pass(c, ones)
        col = pl.multiple_of(base_col + c * CHUNK, CHUNK)
        pltpu.sync_copy(buf, out_hbm.at[:, pl.ds(col, CHUNK)])
        scatter_pass(c, zero)
        return carry

    lax.fori_loop(0, _NCHUNK, chunk_body, 0)


def kernel(input, label_map):
    zeros = jnp.zeros((_ZROWS, CHUNK), jnp.int32)
    run = pl.kernel(
        _sc_onehot_t,
        out_type=jax.ShapeDtypeStruct((VOCAB, BATCH), jnp.int32),
        mesh=plsc.VectorSubcoreMesh(core_axis_name="c", subcore_axis_name="s"),
        compiler_params=pltpu.CompilerParams(
            needs_layout_passes=False, use_tc_tiling_on_sc=True),
        scratch_types=[
            pltpu.VMEM((_COLS_PER_W,), jnp.int32),
            pltpu.VMEM((VOCAB,), jnp.int32),
            pltpu.VMEM((VOCAB, CHUNK), jnp.int32),
            pltpu.VMEM_SHARED((_ZROWS, CHUNK), jnp.int32),
        ],
    )
    return run(input, label_map, zeros).T


# speedup vs baseline: 1.0428x; 1.0049x over previous
import jax
import jax.numpy as jnp
from jax import lax
from jax.experimental import pallas as pl
from jax.experimental.pallas import tpu as pltpu
from jax.experimental.pallas import tpu_sc as plsc

VOCAB = 1000
BATCH = 16384

_info = plsc.get_sparse_core_info()
_NC, _NS, _L = _info.num_cores, _info.num_subcores, _info.num_lanes
_NW = _NC * _NS
_COLS_PER_W = BATCH // _NW
CHUNK = 128
_NCHUNK = _COLS_PER_W // CHUNK
_VPC = CHUNK // _L
_ZROWS = 192


def _sc_onehot_t(inp_hbm, lmap_hbm, zeros_hbm, out_hbm, inp_v, lmap_v, buf,
                 zshared):
    sid = lax.axis_index("s")
    wid = sid * _NC + lax.axis_index("c")
    base_col = wid * _COLS_PER_W

    @pl.when(sid == 0)
    def _():
        pltpu.sync_copy(zeros_hbm, zshared)

    pltpu.sync_copy(inp_hbm.at[pl.ds(base_col, _COLS_PER_W)], inp_v)
    pltpu.sync_copy(lmap_hbm, lmap_v)
    plsc.subcore_barrier()
    for r in range(0, VOCAB, _ZROWS):
        n = min(_ZROWS, VOCAB - r)
        pltpu.sync_copy(zshared.at[pl.ds(0, n)], buf.at[pl.ds(r, n)])

    lane = lax.iota(jnp.int32, _L)
    ones = jnp.full((_L,), 1, jnp.int32)
    zero = jnp.full((_L,), 0, jnp.int32)

    def chunk_body(c, carry):
        ids = []
        for j in range(_VPC):
            raw = inp_v[pl.ds(c * CHUNK + j * _L, _L)]
            ids.append(plsc.load_gather(lmap_v, [raw]))
        for j in range(_VPC):
            plsc.store_scatter(buf, [ids[j], lane + j * _L], ones)
        col = pl.multiple_of(base_col + c * CHUNK, CHUNK)
        pltpu.sync_copy(buf, out_hbm.at[:, pl.ds(col, CHUNK)])
        for j in range(_VPC):
            plsc.store_scatter(buf, [ids[j], lane + j * _L], zero)
        return carry

    lax.fori_loop(0, _NCHUNK, chunk_body, 0)


def kernel(input, label_map):
    zeros = jnp.zeros((_ZROWS, CHUNK), jnp.int32)
    run = pl.kernel(
        _sc_onehot_t,
        out_type=jax.ShapeDtypeStruct((VOCAB, BATCH), jnp.int32),
        mesh=plsc.VectorSubcoreMesh(core_axis_name="c", subcore_axis_name="s"),
        compiler_params=pltpu.CompilerParams(
            needs_layout_passes=False, use_tc_tiling_on_sc=True),
        scratch_types=[
            pltpu.VMEM((_COLS_PER_W,), jnp.int32),
            pltpu.VMEM((VOCAB,), jnp.int32),
            pltpu.VMEM((VOCAB, CHUNK), jnp.int32),
            pltpu.VMEM_SHARED((_ZROWS, CHUNK), jnp.int32),
        ],
    )
    return run(input, label_map, zeros).T


# speedup vs baseline: 1.0475x; 1.0045x over previous
import jax
import jax.numpy as jnp
from jax import lax
from jax.experimental import pallas as pl
from jax.experimental.pallas import tpu as pltpu
from jax.experimental.pallas import tpu_sc as plsc

VOCAB = 1000
BATCH = 16384

_info = plsc.get_sparse_core_info()
_NC, _NS, _L = _info.num_cores, _info.num_subcores, _info.num_lanes
_NW = _NC * _NS
_COLS_PER_W = BATCH // _NW
CHUNK = 128
_NCHUNK = _COLS_PER_W // CHUNK
_VPC = CHUNK // _L
_ZROWS = 192


def _sc_onehot_t(inp_hbm, lmap_hbm, zeros_hbm, out_hbm, inp_v, lmap_v, buf,
                 zshared, fsem):
    sid = lax.axis_index("s")
    wid = sid * _NC + lax.axis_index("c")
    base_col = wid * _COLS_PER_W

    @pl.when(sid == 0)
    def _():
        pltpu.sync_copy(zeros_hbm, zshared)

    pltpu.sync_copy(inp_hbm.at[pl.ds(base_col, _COLS_PER_W)], inp_v)
    pltpu.sync_copy(lmap_hbm, lmap_v)
    plsc.subcore_barrier()
    fans = []
    for r in range(0, VOCAB, _ZROWS):
        n = min(_ZROWS, VOCAB - r)
        fans.append(pltpu.async_copy(zshared.at[pl.ds(0, n)],
                                     buf.at[pl.ds(r, n)], fsem))
    for h in fans:
        h.wait()

    lane = lax.iota(jnp.int32, _L)
    ones = jnp.full((_L,), 1, jnp.int32)
    zero = jnp.full((_L,), 0, jnp.int32)

    def chunk_body(c, carry):
        ids = []
        for j in range(_VPC):
            raw = inp_v[pl.ds(c * CHUNK + j * _L, _L)]
            ids.append(plsc.load_gather(lmap_v, [raw]))
        for j in range(_VPC):
            plsc.store_scatter(buf, [ids[j], lane + j * _L], ones)
        col = pl.multiple_of(base_col + c * CHUNK, CHUNK)
        pltpu.sync_copy(buf, out_hbm.at[:, pl.ds(col, CHUNK)])
        for j in range(_VPC):
            plsc.store_scatter(buf, [ids[j], lane + j * _L], zero)
        return carry

    lax.fori_loop(0, _NCHUNK, chunk_body, 0)


def kernel(input, label_map):
    zeros = jnp.zeros((_ZROWS, CHUNK), jnp.int32)
    run = pl.kernel(
        _sc_onehot_t,
        out_type=jax.ShapeDtypeStruct((VOCAB, BATCH), jnp.int32),
        mesh=plsc.VectorSubcoreMesh(core_axis_name="c", subcore_axis_name="s"),
        compiler_params=pltpu.CompilerParams(
            needs_layout_passes=False, use_tc_tiling_on_sc=True),
        scratch_types=[
            pltpu.VMEM((_COLS_PER_W,), jnp.int32),
            pltpu.VMEM((VOCAB,), jnp.int32),
            pltpu.VMEM((VOCAB, CHUNK), jnp.int32),
            pltpu.VMEM_SHARED((_ZROWS, CHUNK), jnp.int32),
            pltpu.SemaphoreType.DMA,
        ],
    )
    return run(input, label_map, zeros).T


# speedup vs baseline: 1.0730x; 1.0243x over previous
import jax
import jax.numpy as jnp
from jax import lax
from jax.experimental import pallas as pl
from jax.experimental.pallas import tpu as pltpu
from jax.experimental.pallas import tpu_sc as plsc

VOCAB = 1000
BATCH = 16384

_info = plsc.get_sparse_core_info()
_NC, _NS, _L = _info.num_cores, _info.num_subcores, _info.num_lanes
_NW = _NC * _NS
_COLS_PER_W = BATCH // _NW
CHUNK = 128
_NCHUNK = _COLS_PER_W // CHUNK
_VPC = CHUNK // _L
_ZROWS = 192


def _sc_onehot_t(inp_hbm, lmap_hbm, zeros_hbm, out_hbm, inp_v, lmap_v, buf,
                 zshared, fsem, ssem):
    sid = lax.axis_index("s")
    wid = sid * _NC + lax.axis_index("c")
    base_col = wid * _COLS_PER_W

    @pl.when(sid == 0)
    def _():
        pltpu.sync_copy(zeros_hbm, zshared)

    plsc.subcore_barrier()
    stage = [pltpu.async_copy(inp_hbm.at[pl.ds(base_col, _COLS_PER_W)],
                              inp_v, ssem),
             pltpu.async_copy(lmap_hbm, lmap_v, ssem)]
    fans = []
    for r in range(0, VOCAB, _ZROWS):
        n = min(_ZROWS, VOCAB - r)
        fans.append(pltpu.async_copy(zshared.at[pl.ds(0, n)],
                                     buf.at[pl.ds(r, n)], fsem))
    for h in stage + fans:
        h.wait()

    lane = lax.iota(jnp.int32, _L)
    ones = jnp.full((_L,), 1, jnp.int32)
    zero = jnp.full((_L,), 0, jnp.int32)

    def chunk_body(c, carry):
        ids = []
        for j in range(_VPC):
            raw = inp_v[pl.ds(c * CHUNK + j * _L, _L)]
            ids.append(plsc.load_gather(lmap_v, [raw]))
        for j in range(_VPC):
            plsc.store_scatter(buf, [ids[j], lane + j * _L], ones)
        col = pl.multiple_of(base_col + c * CHUNK, CHUNK)
        pltpu.sync_copy(buf, out_hbm.at[:, pl.ds(col, CHUNK)])
        for j in range(_VPC):
            plsc.store_scatter(buf, [ids[j], lane + j * _L], zero)
        return carry

    lax.fori_loop(0, _NCHUNK, chunk_body, 0)


def kernel(input, label_map):
    zeros = jnp.zeros((_ZROWS, CHUNK), jnp.int32)
    run = pl.kernel(
        _sc_onehot_t,
        out_type=jax.ShapeDtypeStruct((VOCAB, BATCH), jnp.int32),
        mesh=plsc.VectorSubcoreMesh(core_axis_name="c", subcore_axis_name="s"),
        compiler_params=pltpu.CompilerParams(
            needs_layout_passes=False, use_tc_tiling_on_sc=True),
        scratch_types=[
            pltpu.VMEM((_COLS_PER_W,), jnp.int32),
            pltpu.VMEM((VOCAB,), jnp.int32),
            pltpu.VMEM((VOCAB, CHUNK), jnp.int32),
            pltpu.VMEM_SHARED((_ZROWS, CHUNK), jnp.int32),
            pltpu.SemaphoreType.DMA,
            pltpu.SemaphoreType.DMA,
        ],
    )
    return run(input, label_map, zeros).T
